# baseline (speedup 1.0000x reference)
import functools

import jax
import jax.numpy as jnp
from jax import lax
from jax.experimental import pallas as pl
from jax.experimental.pallas import tpu as pltpu
from jax.experimental.pallas import tpu_sc as plsc

EMBED_DIM = 64
LANES = 16
VBLK = 128
CHUNK = 320
NBUF = 4


def _mesh_info():
    info = plsc.get_sparse_core_info()
    return info.num_cores, info.num_subcores


VB = 8192


@functools.lru_cache(maxsize=None)
def _build_repack_tc(V: int):
    grid = (V + VB - 1) // VB

    def body(x_ref, o_ref):
        x3 = x_ref[...].T.reshape(VB // 2, 2, EMBED_DIM)
        o_ref[:, 0:EMBED_DIM] = x3[:, 0, :]
        o_ref[:, EMBED_DIM:2 * EMBED_DIM] = x3[:, 1, :]

    return pl.pallas_call(
        body,
        grid=(grid,),
        in_specs=[pl.BlockSpec((EMBED_DIM, VB), lambda i: (0, i))],
        out_specs=pl.BlockSpec((VB // 2, 2 * EMBED_DIM), lambda i: (i, 0)),
        out_shape=jax.ShapeDtypeStruct((V // 2, 2 * EMBED_DIM), jnp.float32),
    )


@functools.lru_cache(maxsize=None)
def _build_repack(V: int):
    NC, NS = _mesh_info()
    NW = NC * NS
    nfull = V // VBLK
    vrem = V - nfull * VBLK
    base = nfull // NW
    extra = nfull % NW
    npairs = (base + (1 if extra else 0) + 1) // 2
    mesh = plsc.VectorSubcoreMesh(core_axis_name="c", subcore_axis_name="s")

    scratch = [pltpu.VMEM((EMBED_DIM, VBLK + 1), jnp.float32) for _ in range(2)]
    scratch += [pltpu.VMEM((VBLK // 2, 2 * EMBED_DIM), jnp.float32)
                for _ in range(2)]
    scratch += [pltpu.SemaphoreType.DMA for _ in range(4)]

    @functools.partial(
        pl.kernel,
        mesh=mesh,
        out_type=jax.ShapeDtypeStruct((V // 2, 2 * EMBED_DIM), jnp.float32),
        scratch_types=scratch,
        compiler_params=pltpu.CompilerParams(needs_layout_passes=False),
    )
    def repack_kernel(tT_hbm, tail_hbm, out_hbm, a0, a1, p0, p1,
                      sr0, sr1, sw0, sw1):
        A = (a0, a1)
        P = (p0, p1)
        sem_r = (sr0, sr1)
        sem_w = (sw0, sw1)
        wid = lax.axis_index("s") * NC + lax.axis_index("c")
        c0 = base * wid + jnp.minimum(wid, extra)
        nblk = base + (wid < extra).astype(jnp.int32)
        cend = c0 + nblk

        def start_r(c, b):
            pltpu.async_copy(tT_hbm.at[:, pl.ds(c * VBLK, VBLK)],
                             A[b].at[:, pl.ds(0, VBLK)], sem_r[b])

        def wait_r(b):
            pltpu.make_async_copy(tT_hbm.at[:, pl.ds(0, VBLK)],
                                  A[b].at[:, pl.ds(0, VBLK)], sem_r[b]).wait()

        def start_w(c, b):
            pltpu.async_copy(P[b], out_hbm.at[pl.ds(c * (VBLK // 2),
                                                    VBLK // 2), :], sem_w[b])

        def wait_w(b):
            pltpu.make_async_copy(P[b], out_hbm.at[pl.ds(0, VBLK // 2), :],
                                  sem_w[b]).wait()

        iota = lax.iota(jnp.int32, LANES)
        rows_k = [iota + (LANES * k) for k in range(EMBED_DIM // LANES)]

        def transpose(b, nq):
            a, p = A[b], P[b]

            def body(q, carry):
                ce = iota * 0 + 2 * q
                co = ce + 1
                for k in range(EMBED_DIM // LANES):
                    p[q, pl.ds(LANES * k, LANES)] = plsc.load_gather(
                        a, [rows_k[k], ce])
                    p[q, pl.ds(EMBED_DIM + LANES * k, LANES)] = (
                        plsc.load_gather(a, [rows_k[k], co]))
                return carry

            lax.fori_loop(0, nq, body, 0)

        start_r(c0, 0)
        start_r(c0 + 1, 1)

        def outer(t, carry):
            for b in range(2):
                c = c0 + 2 * t + b

                @pl.when(c < cend)
                def _():
                    wait_r(b)

                    @pl.when(c >= c0 + 2)
                    def _():
                        wait_w(b)

                    transpose(b, VBLK // 2)
                    start_w(c, b)

                    @pl.when(c + 2 < cend)
                    def _():
                        start_r(c + 2, b)
            return carry

        lax.fori_loop(0, npairs, outer, 0)
        wait_w(0)
        wait_w(1)

        if vrem:
            @pl.when(wid == NW - 1)
            def _():
                pltpu.sync_copy(tail_hbm, A[0].at[:, pl.ds(0, VBLK)])
                transpose(0, vrem // 2)
                pltpu.sync_copy(P[0].at[pl.ds(0, vrem // 2), :],
                                out_hbm.at[pl.ds(nfull * (VBLK // 2),
                                                 vrem // 2), :])

    return repack_kernel


@functools.lru_cache(maxsize=None)
def _build_gather_native(HIST: int, BATCH: int):
    NC, NS = _mesh_info()
    NW = NC * NS
    BW = BATCH // NW
    assert BATCH % NW == 0 and BW % LANES == 0 and HIST % 2 == 0
    mesh = plsc.VectorSubcoreMesh(core_axis_name="c", subcore_axis_name="s")

    GTP = BW + 1
    scratch = [pltpu.VMEM((HIST, BW), jnp.int32)]
    scratch += [pltpu.VMEM((BW, EMBED_DIM), jnp.float32) for _ in range(2)]
    scratch += [pltpu.VMEM((EMBED_DIM, GTP), jnp.float32) for _ in range(2)]
    scratch += [pltpu.SemaphoreType.DMA for _ in range(4)]

    @functools.partial(
        pl.kernel,
        mesh=mesh,
        out_type=jax.ShapeDtypeStruct((HIST, EMBED_DIM, BATCH), jnp.float32),
        scratch_types=scratch,
        compiler_params=pltpu.CompilerParams(use_tc_tiling_on_sc=False,
                                             needs_layout_passes=False),
    )
    def gather_kernel(idx_hbm, table_hbm, out_hbm, idx_v, g0, g1, t0, t1,
                      sg0, sg1, sw0, sw1):
        G = (g0, g1)
        GT = (t0, t1)
        sem_g = (sg0, sg1)
        sem_w = (sw0, sw1)
        wid = lax.axis_index("s") * NC + lax.axis_index("c")
        b0 = wid * BW

        pltpu.sync_copy(idx_hbm.at[:, pl.ds(b0, BW)], idx_v)

        def start_g(l, b):
            pltpu.async_copy(table_hbm.at[idx_v.at[l]], G[b], sem_g[b])

        def wait_g(b):
            pltpu.make_async_copy(table_hbm.at[idx_v.at[0]], G[b],
                                  sem_g[b]).wait()

        def start_w(l, b):
            pltpu.async_copy(GT[b].at[:, pl.ds(0, BW)],
                             out_hbm.at[l, :, pl.ds(b0, BW)], sem_w[b])

        def wait_w(b):
            pltpu.make_async_copy(GT[b].at[:, pl.ds(0, BW)],
                                  out_hbm.at[0, :, pl.ds(b0, BW)],
                                  sem_w[b]).wait()

        iota = lax.iota(jnp.int32, LANES)
        rows_k = [iota + (LANES * k) for k in range(EMBED_DIM // LANES)]

        def transpose(b):
            g, gt = G[b], GT[b]

            def body(gi, carry):
                bi0 = gi * LANES
                for j in range(LANES):
                    bi = bi0 + j
                    cb = iota * 0 + bi
                    for k in range(EMBED_DIM // LANES):
                        vals = g[bi, pl.ds(LANES * k, LANES)]
                        plsc.store_scatter(gt, [rows_k[k], cb], vals)
                return carry

            lax.fori_loop(0, BW // LANES, body, 0)

        start_g(0, 0)
        start_g(1, 1)

        def outer(t, carry):
            for b in range(2):
                l = 2 * t + b
                wait_g(b)

                @pl.when(l >= 2)
                def _():
                    wait_w(b)

                transpose(b)
                start_w(l, b)

                @pl.when(l + 2 < HIST)
                def _():
                    start_g(l + 2, b)
            return carry

        lax.fori_loop(0, HIST // 2, outer, 0)
        wait_w(0)
        wait_w(1)

    return gather_kernel


@functools.lru_cache(maxsize=None)
def _build_gather(B: int, V: int):
    NC, NS = _mesh_info()
    NW = NC * NS
    b_per_w = B // NW
    nsteps = b_per_w // CHUNK
    assert B % NW == 0 and b_per_w % CHUNK == 0 and nsteps % NBUF == 0
    mesh = plsc.VectorSubcoreMesh(core_axis_name="c", subcore_axis_name="s")

    scratch = [pltpu.VMEM((b_per_w,), jnp.int32)]
    scratch += [pltpu.VMEM((CHUNK, EMBED_DIM), jnp.float32)
                for _ in range(NBUF)]
    scratch += [pltpu.SemaphoreType.DMA for _ in range(2 * NBUF)]

    @functools.partial(
        pl.kernel,
        mesh=mesh,
        out_type=jax.ShapeDtypeStruct((B, EMBED_DIM), jnp.float32),
        scratch_types=scratch,
        compiler_params=pltpu.CompilerParams(use_tc_tiling_on_sc=False),
    )
    def gather_kernel(idx_hbm, table_hbm, out_hbm, idx_v, *rest):
        rows = rest[:NBUF]
        sem_g = rest[NBUF:2 * NBUF]
        sem_w = rest[2 * NBUF:]
        wid = lax.axis_index("s") * NC + lax.axis_index("c")
        bbase = wid * b_per_w

        pltpu.sync_copy(idx_hbm.at[pl.ds(bbase, b_per_w)], idx_v)

        def start_g(s, b):
            pltpu.async_copy(
                table_hbm.at[idx_v.at[pl.ds(s * CHUNK, CHUNK)]], rows[b],
                sem_g[b])

        def wait_g(b):
            pltpu.make_async_copy(
                table_hbm.at[idx_v.at[pl.ds(0, CHUNK)]], rows[b],
                sem_g[b]).wait()

        def start_w(s, b):
            pltpu.async_copy(
                rows[b], out_hbm.at[pl.ds(bbase + s * CHUNK, CHUNK)],
                sem_w[b])

        def wait_w(b):
            pltpu.make_async_copy(
                rows[b], out_hbm.at[pl.ds(bbase, CHUNK)], sem_w[b]).wait()

        start_g(0, 0)
        start_g(1, 1)

        def outer(t, carry):
            for b in range(NBUF):
                s = t * NBUF + b
                wait_g(b)
                start_w(s, b)
                b2 = (b + 2) % NBUF

                @pl.when(s + 2 < nsteps)
                def _issue():
                    @pl.when(s >= 2)
                    def _drain():
                        wait_w(b2)
                    start_g(s + 2, b2)
            return carry

        lax.fori_loop(0, nsteps // NBUF, outer, 0)
        for b in range(NBUF):
            wait_w(b)

    return gather_kernel


def kernel(input, hidden, table):
    BATCH, HIST = input.shape
    V, E = table.shape
    B = BATCH * HIST
    packed = _build_repack_tc(V)(table.T)
    t_rm = packed.reshape(V, E)
    idx2d = input.T.astype(jnp.int32)
    out = _build_gather_native(HIST, BATCH)(idx2d, t_rm)
    return out.transpose(2, 0, 1)


# speedup vs baseline: 1.4904x; 1.4904x over previous
import functools

import jax
import jax.numpy as jnp
from jax import lax
from jax.experimental import pallas as pl
from jax.experimental.pallas import tpu as pltpu
from jax.experimental.pallas import tpu_sc as plsc

EMBED_DIM = 64
LANES = 16
VBLK = 128
CHUNK = 320
NBUF = 4


def _mesh_info():
    info = plsc.get_sparse_core_info()
    return info.num_cores, info.num_subcores


VB = 8192


@functools.lru_cache(maxsize=None)
def _build_repack_tc(V: int):
    grid = (V + VB - 1) // VB

    def body(x_ref, o_ref):
        x3 = x_ref[...].T.reshape(VB // 2, 2, EMBED_DIM)
        o_ref[:, 0:EMBED_DIM] = x3[:, 0, :]
        o_ref[:, EMBED_DIM:2 * EMBED_DIM] = x3[:, 1, :]

    return pl.pallas_call(
        body,
        grid=(grid,),
        in_specs=[pl.BlockSpec((EMBED_DIM, VB), lambda i: (0, i))],
        out_specs=pl.BlockSpec((VB // 2, 2 * EMBED_DIM), lambda i: (i, 0)),
        out_shape=jax.ShapeDtypeStruct((V // 2, 2 * EMBED_DIM), jnp.float32),
    )


@functools.lru_cache(maxsize=None)
def _build_repack(V: int):
    NC, NS = _mesh_info()
    NW = NC * NS
    nfull = V // VBLK
    vrem = V - nfull * VBLK
    base = nfull // NW
    extra = nfull % NW
    npairs = (base + (1 if extra else 0) + 1) // 2
    mesh = plsc.VectorSubcoreMesh(core_axis_name="c", subcore_axis_name="s")

    scratch = [pltpu.VMEM((EMBED_DIM, VBLK + 1), jnp.float32) for _ in range(2)]
    scratch += [pltpu.VMEM((VBLK // 2, 2 * EMBED_DIM), jnp.float32)
                for _ in range(2)]
    scratch += [pltpu.SemaphoreType.DMA for _ in range(4)]

    @functools.partial(
        pl.kernel,
        mesh=mesh,
        out_type=jax.ShapeDtypeStruct((V // 2, 2 * EMBED_DIM), jnp.float32),
        scratch_types=scratch,
        compiler_params=pltpu.CompilerParams(needs_layout_passes=False),
    )
    def repack_kernel(tT_hbm, tail_hbm, out_hbm, a0, a1, p0, p1,
                      sr0, sr1, sw0, sw1):
        A = (a0, a1)
        P = (p0, p1)
        sem_r = (sr0, sr1)
        sem_w = (sw0, sw1)
        wid = lax.axis_index("s") * NC + lax.axis_index("c")
        c0 = base * wid + jnp.minimum(wid, extra)
        nblk = base + (wid < extra).astype(jnp.int32)
        cend = c0 + nblk

        def start_r(c, b):
            pltpu.async_copy(tT_hbm.at[:, pl.ds(c * VBLK, VBLK)],
                             A[b].at[:, pl.ds(0, VBLK)], sem_r[b])

        def wait_r(b):
            pltpu.make_async_copy(tT_hbm.at[:, pl.ds(0, VBLK)],
                                  A[b].at[:, pl.ds(0, VBLK)], sem_r[b]).wait()

        def start_w(c, b):
            pltpu.async_copy(P[b], out_hbm.at[pl.ds(c * (VBLK // 2),
                                                    VBLK // 2), :], sem_w[b])

        def wait_w(b):
            pltpu.make_async_copy(P[b], out_hbm.at[pl.ds(0, VBLK // 2), :],
                                  sem_w[b]).wait()

        iota = lax.iota(jnp.int32, LANES)
        rows_k = [iota + (LANES * k) for k in range(EMBED_DIM // LANES)]

        def transpose(b, nq):
            a, p = A[b], P[b]

            def body(q, carry):
                ce = iota * 0 + 2 * q
                co = ce + 1
                for k in range(EMBED_DIM // LANES):
                    p[q, pl.ds(LANES * k, LANES)] = plsc.load_gather(
                        a, [rows_k[k], ce])
                    p[q, pl.ds(EMBED_DIM + LANES * k, LANES)] = (
                        plsc.load_gather(a, [rows_k[k], co]))
                return carry

            lax.fori_loop(0, nq, body, 0)

        start_r(c0, 0)
        start_r(c0 + 1, 1)

        def outer(t, carry):
            for b in range(2):
                c = c0 + 2 * t + b

                @pl.when(c < cend)
                def _():
                    wait_r(b)

                    @pl.when(c >= c0 + 2)
                    def _():
                        wait_w(b)

                    transpose(b, VBLK // 2)
                    start_w(c, b)

                    @pl.when(c + 2 < cend)
                    def _():
                        start_r(c + 2, b)
            return carry

        lax.fori_loop(0, npairs, outer, 0)
        wait_w(0)
        wait_w(1)

        if vrem:
            @pl.when(wid == NW - 1)
            def _():
                pltpu.sync_copy(tail_hbm, A[0].at[:, pl.ds(0, VBLK)])
                transpose(0, vrem // 2)
                pltpu.sync_copy(P[0].at[pl.ds(0, vrem // 2), :],
                                out_hbm.at[pl.ds(nfull * (VBLK // 2),
                                                 vrem // 2), :])

    return repack_kernel


@functools.lru_cache(maxsize=None)
def _build_gather_native(HIST: int, BATCH: int):
    NC, NS = _mesh_info()
    NW = NC * NS
    BW = BATCH // NW
    assert BATCH % NW == 0 and BW % LANES == 0 and HIST % 2 == 0
    mesh = plsc.VectorSubcoreMesh(core_axis_name="c", subcore_axis_name="s")

    GTP = BW + 1
    scratch = [pltpu.VMEM((HIST, BW), jnp.int32)]
    scratch += [pltpu.VMEM((BW, EMBED_DIM), jnp.float32) for _ in range(2)]
    scratch += [pltpu.VMEM((EMBED_DIM, GTP), jnp.float32) for _ in range(2)]
    scratch += [pltpu.SemaphoreType.DMA for _ in range(4)]

    @functools.partial(
        pl.kernel,
        mesh=mesh,
        out_type=jax.ShapeDtypeStruct((HIST, EMBED_DIM, BATCH), jnp.float32),
        scratch_types=scratch,
        compiler_params=pltpu.CompilerParams(use_tc_tiling_on_sc=False,
                                             needs_layout_passes=False),
    )
    def gather_kernel(idx_hbm, table_hbm, out_hbm, idx_v, g0, g1, t0, t1,
                      sg0, sg1, sw0, sw1):
        G = (g0, g1)
        GT = (t0, t1)
        sem_g = (sg0, sg1)
        sem_w = (sw0, sw1)
        wid = lax.axis_index("s") * NC + lax.axis_index("c")
        b0 = wid * BW

        pltpu.sync_copy(idx_hbm.at[:, pl.ds(b0, BW)], idx_v)

        def start_g(l, b):
            pltpu.async_copy(table_hbm.at[idx_v.at[l]], G[b], sem_g[b])

        def wait_g(b):
            pltpu.make_async_copy(table_hbm.at[idx_v.at[0]], G[b],
                                  sem_g[b]).wait()

        def start_w(l, b):
            pltpu.async_copy(GT[b].at[:, pl.ds(0, BW)],
                             out_hbm.at[l, :, pl.ds(b0, BW)], sem_w[b])

        def wait_w(b):
            pltpu.make_async_copy(GT[b].at[:, pl.ds(0, BW)],
                                  out_hbm.at[0, :, pl.ds(b0, BW)],
                                  sem_w[b]).wait()

        iota = lax.iota(jnp.int32, LANES)
        rows_k = [iota + (LANES * k) for k in range(EMBED_DIM // LANES)]

        def transpose(b):
            g, gt = G[b], GT[b]

            def body(gi, carry):
                bi0 = gi * LANES
                for j in range(LANES):
                    bi = bi0 + j
                    cb = iota * 0 + bi
                    for k in range(EMBED_DIM // LANES):
                        vals = g[bi, pl.ds(LANES * k, LANES)]
                        plsc.store_scatter(gt, [rows_k[k], cb], vals)
                return carry

            lax.fori_loop(0, BW // LANES, body, 0)

        start_g(0, 0)
        start_g(1, 1)

        def outer(t, carry):
            for b in range(2):
                l = 2 * t + b
                wait_g(b)

                @pl.when(l >= 2)
                def _():
                    wait_w(b)

                transpose(b)
                start_w(l, b)

                @pl.when(l + 2 < HIST)
                def _():
                    start_g(l + 2, b)
            return carry

        lax.fori_loop(0, HIST // 2, outer, 0)
        wait_w(0)
        wait_w(1)

    return gather_kernel


@functools.lru_cache(maxsize=None)
def _build_gather(B: int, V: int):
    NC, NS = _mesh_info()
    NW = NC * NS
    b_per_w = B // NW
    nsteps = b_per_w // CHUNK
    assert B % NW == 0 and b_per_w % CHUNK == 0 and nsteps % NBUF == 0
    mesh = plsc.VectorSubcoreMesh(core_axis_name="c", subcore_axis_name="s")

    scratch = [pltpu.VMEM((b_per_w,), jnp.int32)]
    scratch += [pltpu.VMEM((CHUNK, EMBED_DIM), jnp.float32)
                for _ in range(NBUF)]
    scratch += [pltpu.SemaphoreType.DMA for _ in range(2 * NBUF)]

    @functools.partial(
        pl.kernel,
        mesh=mesh,
        out_type=jax.ShapeDtypeStruct((B, 2 * EMBED_DIM), jnp.float32),
        scratch_types=scratch,
        compiler_params=pltpu.CompilerParams(use_tc_tiling_on_sc=False),
    )
    def gather_kernel(idx_hbm, table_hbm, out_hbm, idx_v, *rest):
        rows = rest[:NBUF]
        sem_g = rest[NBUF:2 * NBUF]
        sem_w = rest[2 * NBUF:]
        wid = lax.axis_index("s") * NC + lax.axis_index("c")
        bbase = wid * b_per_w

        pltpu.sync_copy(idx_hbm.at[pl.ds(bbase, b_per_w)], idx_v)

        def start_g(s, b):
            pltpu.async_copy(
                table_hbm.at[idx_v.at[pl.ds(s * CHUNK, CHUNK)]], rows[b],
                sem_g[b])

        def wait_g(b):
            pltpu.make_async_copy(
                table_hbm.at[idx_v.at[pl.ds(0, CHUNK)]], rows[b],
                sem_g[b]).wait()

        def start_w(s, b):
            pltpu.async_copy(
                rows[b],
                out_hbm.at[pl.ds(bbase + s * CHUNK, CHUNK),
                           pl.ds(0, EMBED_DIM)], sem_w[b])

        def wait_w(b):
            pltpu.make_async_copy(
                rows[b],
                out_hbm.at[pl.ds(bbase, CHUNK), pl.ds(0, EMBED_DIM)],
                sem_w[b]).wait()

        start_g(0, 0)
        start_g(1, 1)

        def outer(t, carry):
            for b in range(NBUF):
                s = t * NBUF + b
                wait_g(b)
                start_w(s, b)
                b2 = (b + 2) % NBUF

                @pl.when(s + 2 < nsteps)
                def _issue():
                    @pl.when(s >= 2)
                    def _drain():
                        wait_w(b2)
                    start_g(s + 2, b2)
            return carry

        lax.fori_loop(0, nsteps // NBUF, outer, 0)
        for b in range(NBUF):
            wait_w(b)

    return gather_kernel


def kernel(input, hidden, table):
    BATCH, HIST = input.shape
    V, E = table.shape
    B = BATCH * HIST
    packed = _build_repack_tc(V)(table.T)
    t_rm = packed.reshape(V, E)
    idx = input.T.reshape(B).astype(jnp.int32)
    out = _build_gather(B, V)(idx, t_rm)
    return out.reshape(HIST, BATCH, 2 * E)[:, :, :E].transpose(1, 0, 2)


# speedup vs baseline: 1.5061x; 1.0105x over previous
import functools

import jax
import jax.numpy as jnp
from jax import lax
from jax.experimental import pallas as pl
from jax.experimental.pallas import tpu as pltpu
from jax.experimental.pallas import tpu_sc as plsc

EMBED_DIM = 64
LANES = 16
VBLK = 128
CHUNK = 320
NBUF = 4


def _mesh_info():
    info = plsc.get_sparse_core_info()
    return info.num_cores, info.num_subcores


VB = 16384


@functools.lru_cache(maxsize=None)
def _build_repack_tc(V: int):
    grid = (V + VB - 1) // VB

    def body(x_ref, o_ref):
        x3 = x_ref[...].T.reshape(VB // 2, 2, EMBED_DIM)
        o_ref[:, 0:EMBED_DIM] = x3[:, 0, :]
        o_ref[:, EMBED_DIM:2 * EMBED_DIM] = x3[:, 1, :]

    return pl.pallas_call(
        body,
        grid=(grid,),
        in_specs=[pl.BlockSpec((EMBED_DIM, VB), lambda i: (0, i))],
        out_specs=pl.BlockSpec((VB // 2, 2 * EMBED_DIM), lambda i: (i, 0)),
        out_shape=jax.ShapeDtypeStruct((V // 2, 2 * EMBED_DIM), jnp.float32),
    )


@functools.lru_cache(maxsize=None)
def _build_repack(V: int):
    NC, NS = _mesh_info()
    NW = NC * NS
    nfull = V // VBLK
    vrem = V - nfull * VBLK
    base = nfull // NW
    extra = nfull % NW
    npairs = (base + (1 if extra else 0) + 1) // 2
    mesh = plsc.VectorSubcoreMesh(core_axis_name="c", subcore_axis_name="s")

    scratch = [pltpu.VMEM((EMBED_DIM, VBLK + 1), jnp.float32) for _ in range(2)]
    scratch += [pltpu.VMEM((VBLK // 2, 2 * EMBED_DIM), jnp.float32)
                for _ in range(2)]
    scratch += [pltpu.SemaphoreType.DMA for _ in range(4)]

    @functools.partial(
        pl.kernel,
        mesh=mesh,
        out_type=jax.ShapeDtypeStruct((V // 2, 2 * EMBED_DIM), jnp.float32),
        scratch_types=scratch,
        compiler_params=pltpu.CompilerParams(needs_layout_passes=False),
    )
    def repack_kernel(tT_hbm, tail_hbm, out_hbm, a0, a1, p0, p1,
                      sr0, sr1, sw0, sw1):
        A = (a0, a1)
        P = (p0, p1)
        sem_r = (sr0, sr1)
        sem_w = (sw0, sw1)
        wid = lax.axis_index("s") * NC + lax.axis_index("c")
        c0 = base * wid + jnp.minimum(wid, extra)
        nblk = base + (wid < extra).astype(jnp.int32)
        cend = c0 + nblk

        def start_r(c, b):
            pltpu.async_copy(tT_hbm.at[:, pl.ds(c * VBLK, VBLK)],
                             A[b].at[:, pl.ds(0, VBLK)], sem_r[b])

        def wait_r(b):
            pltpu.make_async_copy(tT_hbm.at[:, pl.ds(0, VBLK)],
                                  A[b].at[:, pl.ds(0, VBLK)], sem_r[b]).wait()

        def start_w(c, b):
            pltpu.async_copy(P[b], out_hbm.at[pl.ds(c * (VBLK // 2),
                                                    VBLK // 2), :], sem_w[b])

        def wait_w(b):
            pltpu.make_async_copy(P[b], out_hbm.at[pl.ds(0, VBLK // 2), :],
                                  sem_w[b]).wait()

        iota = lax.iota(jnp.int32, LANES)
        rows_k = [iota + (LANES * k) for k in range(EMBED_DIM // LANES)]

        def transpose(b, nq):
            a, p = A[b], P[b]

            def body(q, carry):
                ce = iota * 0 + 2 * q
                co = ce + 1
                for k in range(EMBED_DIM // LANES):
                    p[q, pl.ds(LANES * k, LANES)] = plsc.load_gather(
                        a, [rows_k[k], ce])
                    p[q, pl.ds(EMBED_DIM + LANES * k, LANES)] = (
                        plsc.load_gather(a, [rows_k[k], co]))
                return carry

            lax.fori_loop(0, nq, body, 0)

        start_r(c0, 0)
        start_r(c0 + 1, 1)

        def outer(t, carry):
            for b in range(2):
                c = c0 + 2 * t + b

                @pl.when(c < cend)
                def _():
                    wait_r(b)

                    @pl.when(c >= c0 + 2)
                    def _():
                        wait_w(b)

                    transpose(b, VBLK // 2)
                    start_w(c, b)

                    @pl.when(c + 2 < cend)
                    def _():
                        start_r(c + 2, b)
            return carry

        lax.fori_loop(0, npairs, outer, 0)
        wait_w(0)
        wait_w(1)

        if vrem:
            @pl.when(wid == NW - 1)
            def _():
                pltpu.sync_copy(tail_hbm, A[0].at[:, pl.ds(0, VBLK)])
                transpose(0, vrem // 2)
                pltpu.sync_copy(P[0].at[pl.ds(0, vrem // 2), :],
                                out_hbm.at[pl.ds(nfull * (VBLK // 2),
                                                 vrem // 2), :])

    return repack_kernel


@functools.lru_cache(maxsize=None)
def _build_gather_native(HIST: int, BATCH: int):
    NC, NS = _mesh_info()
    NW = NC * NS
    BW = BATCH // NW
    assert BATCH % NW == 0 and BW % LANES == 0 and HIST % 2 == 0
    mesh = plsc.VectorSubcoreMesh(core_axis_name="c", subcore_axis_name="s")

    GTP = BW + 1
    scratch = [pltpu.VMEM((HIST, BW), jnp.int32)]
    scratch += [pltpu.VMEM((BW, EMBED_DIM), jnp.float32) for _ in range(2)]
    scratch += [pltpu.VMEM((EMBED_DIM, GTP), jnp.float32) for _ in range(2)]
    scratch += [pltpu.SemaphoreType.DMA for _ in range(4)]

    @functools.partial(
        pl.kernel,
        mesh=mesh,
        out_type=jax.ShapeDtypeStruct((HIST, EMBED_DIM, BATCH), jnp.float32),
        scratch_types=scratch,
        compiler_params=pltpu.CompilerParams(use_tc_tiling_on_sc=False,
                                             needs_layout_passes=False),
    )
    def gather_kernel(idx_hbm, table_hbm, out_hbm, idx_v, g0, g1, t0, t1,
                      sg0, sg1, sw0, sw1):
        G = (g0, g1)
        GT = (t0, t1)
        sem_g = (sg0, sg1)
        sem_w = (sw0, sw1)
        wid = lax.axis_index("s") * NC + lax.axis_index("c")
        b0 = wid * BW

        pltpu.sync_copy(idx_hbm.at[:, pl.ds(b0, BW)], idx_v)

        def start_g(l, b):
            pltpu.async_copy(table_hbm.at[idx_v.at[l]], G[b], sem_g[b])

        def wait_g(b):
            pltpu.make_async_copy(table_hbm.at[idx_v.at[0]], G[b],
                                  sem_g[b]).wait()

        def start_w(l, b):
            pltpu.async_copy(GT[b].at[:, pl.ds(0, BW)],
                             out_hbm.at[l, :, pl.ds(b0, BW)], sem_w[b])

        def wait_w(b):
            pltpu.make_async_copy(GT[b].at[:, pl.ds(0, BW)],
                                  out_hbm.at[0, :, pl.ds(b0, BW)],
                                  sem_w[b]).wait()

        iota = lax.iota(jnp.int32, LANES)
        rows_k = [iota + (LANES * k) for k in range(EMBED_DIM // LANES)]

        def transpose(b):
            g, gt = G[b], GT[b]

            def body(gi, carry):
                bi0 = gi * LANES
                for j in range(LANES):
                    bi = bi0 + j
                    cb = iota * 0 + bi
                    for k in range(EMBED_DIM // LANES):
                        vals = g[bi, pl.ds(LANES * k, LANES)]
                        plsc.store_scatter(gt, [rows_k[k], cb], vals)
                return carry

            lax.fori_loop(0, BW // LANES, body, 0)

        start_g(0, 0)
        start_g(1, 1)

        def outer(t, carry):
            for b in range(2):
                l = 2 * t + b
                wait_g(b)

                @pl.when(l >= 2)
                def _():
                    wait_w(b)

                transpose(b)
                start_w(l, b)

                @pl.when(l + 2 < HIST)
                def _():
                    start_g(l + 2, b)
            return carry

        lax.fori_loop(0, HIST // 2, outer, 0)
        wait_w(0)
        wait_w(1)

    return gather_kernel


@functools.lru_cache(maxsize=None)
def _build_gather(B: int, V: int):
    NC, NS = _mesh_info()
    NW = NC * NS
    b_per_w = B // NW
    nsteps = b_per_w // CHUNK
    assert B % NW == 0 and b_per_w % CHUNK == 0 and nsteps % NBUF == 0
    mesh = plsc.VectorSubcoreMesh(core_axis_name="c", subcore_axis_name="s")

    scratch = [pltpu.VMEM((b_per_w,), jnp.int32)]
    scratch += [pltpu.VMEM((CHUNK, EMBED_DIM), jnp.float32)
                for _ in range(NBUF)]
    scratch += [pltpu.SemaphoreType.DMA for _ in range(2 * NBUF)]

    @functools.partial(
        pl.kernel,
        mesh=mesh,
        out_type=jax.ShapeDtypeStruct((B, 2 * EMBED_DIM), jnp.float32),
        scratch_types=scratch,
        compiler_params=pltpu.CompilerParams(use_tc_tiling_on_sc=False),
    )
    def gather_kernel(idx_hbm, table_hbm, out_hbm, idx_v, *rest):
        rows = rest[:NBUF]
        sem_g = rest[NBUF:2 * NBUF]
        sem_w = rest[2 * NBUF:]
        wid = lax.axis_index("s") * NC + lax.axis_index("c")
        bbase = wid * b_per_w

        pltpu.sync_copy(idx_hbm.at[pl.ds(bbase, b_per_w)], idx_v)

        def start_g(s, b):
            pltpu.async_copy(
                table_hbm.at[idx_v.at[pl.ds(s * CHUNK, CHUNK)]], rows[b],
                sem_g[b])

        def wait_g(b):
            pltpu.make_async_copy(
                table_hbm.at[idx_v.at[pl.ds(0, CHUNK)]], rows[b],
                sem_g[b]).wait()

        def start_w(s, b):
            pltpu.async_copy(
                rows[b],
                out_hbm.at[pl.ds(bbase + s * CHUNK, CHUNK),
                           pl.ds(0, EMBED_DIM)], sem_w[b])

        def wait_w(b):
            pltpu.make_async_copy(
                rows[b],
                out_hbm.at[pl.ds(bbase, CHUNK), pl.ds(0, EMBED_DIM)],
                sem_w[b]).wait()

        start_g(0, 0)
        start_g(1, 1)

        def outer(t, carry):
            for b in range(NBUF):
                s = t * NBUF + b
                wait_g(b)
                start_w(s, b)
                b2 = (b + 2) % NBUF

                @pl.when(s + 2 < nsteps)
                def _issue():
                    @pl.when(s >= 2)
                    def _drain():
                        wait_w(b2)
                    start_g(s + 2, b2)
            return carry

        lax.fori_loop(0, nsteps // NBUF, outer, 0)
        for b in range(NBUF):
            wait_w(b)

    return gather_kernel


def kernel(input, hidden, table):
    BATCH, HIST = input.shape
    V, E = table.shape
    B = BATCH * HIST
    packed = _build_repack_tc(V)(table.T)
    t_rm = packed.reshape(V, E)
    idx = input.T.reshape(B).astype(jnp.int32)
    out = _build_gather(B, V)(idx, t_rm)
    return out.reshape(HIST, BATCH, 2 * E)[:, :, :E].transpose(1, 0, 2)
